# baseline (speedup 1.0000x reference)
import functools

import jax
import jax.numpy as jnp
from jax import lax
from jax.experimental import pallas as pl
from jax.experimental.pallas import tpu as pltpu
from jax.experimental.pallas import tpu_sc as plsc

VOCAB = 1000000
EMBED_DIM = 64
BATCH = 4096
HIST = 200
B = BATCH * HIST

_INFO = plsc.get_sparse_core_info()
NC = _INFO.num_cores
NS = _INFO.num_subcores
NW = NC * NS

NBLK = (VOCAB + 127) // 128
DENSE_ROWS = NBLK * 64
BLK_PER_W = (NBLK + NW - 1) // NW


@functools.partial(
    pl.kernel,
    out_type=jax.ShapeDtypeStruct((DENSE_ROWS, 128), jnp.float32),
    mesh=plsc.VectorSubcoreMesh(core_axis_name="c", subcore_axis_name="s"),
    scratch_types=[
        pltpu.VMEM((2, 64, 128), jnp.float32),
        pltpu.VMEM((2, 64, 128), jnp.float32),
        pltpu.SemaphoreType.DMA((2,)),
        pltpu.SemaphoreType.DMA((2,)),
    ],
    compiler_params=pltpu.CompilerParams(
        use_tc_tiling_on_sc=True, needs_layout_passes=False),
)
def _table_transpose(tableT_hbm, dense_hbm, src_v, dst_v, sem_i, sem_o):
    w = lax.axis_index("s") * NC + lax.axis_index("c")
    j0 = w * BLK_PER_W

    idx_ds = [lax.iota(jnp.int32, 16) + 16 * k for k in range(4)]

    def start_in(jj, b):
        j = j0 + jj

        @pl.when((jj < BLK_PER_W) & (j < NBLK))
        def _():
            pltpu.async_copy(
                tableT_hbm.at[:, pl.ds(j * 128, 128)], src_v.at[b], sem_i.at[b])

    def wait_in(b):
        pltpu.make_async_copy(
            tableT_hbm.at[:, pl.ds(0, 128)], src_v.at[b], sem_i.at[b]).wait()

    def wait_out(b):
        pltpu.make_async_copy(
            dst_v.at[b], dense_hbm.at[pl.ds(0, 64)], sem_o.at[b]).wait()

    def transpose(b):
        def row(q, carry):
            for c2 in range(2):
                idx_c = jnp.full((16,), 2 * q + c2, jnp.int32)
                for k in range(4):
                    vals = plsc.load_gather(src_v.at[b], [idx_ds[k], idx_c])
                    dst_v[b, q, pl.ds(c2 * 64 + 16 * k, 16)] = vals
            return carry

        lax.fori_loop(0, 64, row, 0)

    start_in(0, 0)
    start_in(1, 1)

    def outer(g, carry):
        for b in range(2):
            jj = g * 2 + b
            j = j0 + jj

            @pl.when((jj < BLK_PER_W) & (j < NBLK))
            def _():
                wait_in(b)

                @pl.when(jj >= 2)
                def _():
                    wait_out(b)

                transpose(b)
                start_in(jj + 2, b)
                pltpu.async_copy(
                    dst_v.at[b], dense_hbm.at[pl.ds(j * 64, 64)], sem_o.at[b])
        return carry

    lax.fori_loop(0, (BLK_PER_W + 1) // 2, outer, 0)

    for b in range(2):
        jj_last = BLK_PER_W - 2 + b

        @pl.when((jj_last >= 0) & (j0 + jj_last < NBLK))
        def _():
            wait_out(b)


NBUF = 2
CHUNK = 800
NCHUNK = (B // NW) // CHUNK
BPW = B // NW


@functools.partial(
    pl.kernel,
    out_type=jax.ShapeDtypeStruct((B, 128), jnp.float32),
    mesh=plsc.VectorSubcoreMesh(core_axis_name="c", subcore_axis_name="s"),
    scratch_types=[
        pltpu.VMEM((NBUF, CHUNK), jnp.int32),
        pltpu.VMEM((NBUF, CHUNK, EMBED_DIM), jnp.float32),
        pltpu.SemaphoreType.DMA((NBUF,)),
        pltpu.SemaphoreType.DMA((NBUF,)),
        pltpu.SemaphoreType.DMA((NBUF,)),
    ],
    compiler_params=pltpu.CompilerParams(use_tc_tiling_on_sc=False),
)
def _embed_gather(doc_hbm, table_hbm, out_hbm, idx_v, rows_v, sem_i, sem_g, sem_s):
    wid = lax.axis_index("s") * NC + lax.axis_index("c")
    base = wid * BPW

    def start_idx(c, b):
        pltpu.async_copy(
            doc_hbm.at[pl.ds(base + c * CHUNK, CHUNK)], idx_v.at[b], sem_i.at[b])

    def wait_idx(b):
        pltpu.make_async_copy(
            doc_hbm.at[pl.ds(0, CHUNK)], idx_v.at[b], sem_i.at[b]).wait()

    def wait_scatter(b):
        pltpu.make_async_copy(
            rows_v.at[b],
            out_hbm.at[pl.ds(0, CHUNK), pl.ds(0, EMBED_DIM)],
            sem_s.at[b]).wait()

    for b in range(NBUF):
        start_idx(b, b)

    def outer(g, carry):
        for b in range(NBUF):
            c = g * NBUF + b

            @pl.when(c >= NBUF)
            def _():
                wait_scatter(b)

            wait_idx(b)
            pltpu.async_copy(
                table_hbm.at[idx_v.at[b]], rows_v.at[b], sem_g.at[b]).wait()

            @pl.when(c + NBUF < NCHUNK)
            def _():
                start_idx(c + NBUF, b)

            pltpu.async_copy(
                rows_v.at[b],
                out_hbm.at[pl.ds(base + c * CHUNK, CHUNK), pl.ds(0, EMBED_DIM)],
                sem_s.at[b])
        return carry

    lax.fori_loop(0, NCHUNK // NBUF, outer, 0)

    for b in range(NBUF):
        wait_scatter(b)


def kernel(doc, table):
    flat = doc.reshape(B).astype(jnp.int32)
    dense = _table_transpose(table.T)
    dense_rows = dense.reshape(DENSE_ROWS * 2, EMBED_DIM)
    out = _embed_gather(flat, dense_rows)
    return out[:, :EMBED_DIM].reshape(BATCH, HIST, EMBED_DIM)


# speedup vs baseline: 2.0855x; 2.0855x over previous
import functools

import jax
import jax.numpy as jnp
from jax import lax
from jax.experimental import pallas as pl
from jax.experimental.pallas import tpu as pltpu
from jax.experimental.pallas import tpu_sc as plsc

VOCAB = 1000000
EMBED_DIM = 64
BATCH = 4096
HIST = 200
B = BATCH * HIST

_INFO = plsc.get_sparse_core_info()
NC = _INFO.num_cores
NS = _INFO.num_subcores
NW = NC * NS

NBLK = (VOCAB + 127) // 128
DENSE_ROWS = NBLK * 64
BLK_PER_W = (NBLK + NW - 1) // NW


@functools.partial(
    pl.kernel,
    out_type=jax.ShapeDtypeStruct((DENSE_ROWS, 128), jnp.float32),
    mesh=plsc.VectorSubcoreMesh(core_axis_name="c", subcore_axis_name="s"),
    scratch_types=[
        pltpu.VMEM((2, 64, 128), jnp.float32),
        pltpu.VMEM((2, 64, 128), jnp.float32),
        pltpu.SemaphoreType.DMA((2,)),
        pltpu.SemaphoreType.DMA((2,)),
    ],
    compiler_params=pltpu.CompilerParams(
        use_tc_tiling_on_sc=True, needs_layout_passes=False),
)
def _table_transpose(tableT_hbm, dense_hbm, src_v, dst_v, sem_i, sem_o):
    w = lax.axis_index("s") * NC + lax.axis_index("c")
    j0 = w * BLK_PER_W

    iota = lax.iota(jnp.int32, 16)

    def start_in(jj, b):
        j = j0 + jj

        @pl.when((jj < BLK_PER_W) & (j < NBLK))
        def _():
            pltpu.async_copy(
                tableT_hbm.at[:, pl.ds(j * 128, 128)], src_v.at[b], sem_i.at[b])

    def wait_in(b):
        pltpu.make_async_copy(
            tableT_hbm.at[:, pl.ds(0, 128)], src_v.at[b], sem_i.at[b]).wait()

    def wait_out(b):
        pltpu.make_async_copy(
            dst_v.at[b], dense_hbm.at[pl.ds(0, 64)], sem_o.at[b]).wait()

    def transpose(b):
        def diag(c0, carry):
            cv = c0 + iota
            cv = jnp.where(cv >= 128, cv - 128, cv)
            qv = lax.shift_right_logical(cv, 1)
            c2v = lax.shift_left(jnp.bitwise_and(cv, 1), 6)
            for g in range(4):
                dv = 16 * g + iota
                vals = plsc.load_gather(src_v.at[b], [dv, cv])
                plsc.store_scatter(dst_v.at[b], [qv, c2v + dv], vals)
            return carry

        lax.fori_loop(0, 128, diag, 0)

    start_in(0, 0)
    start_in(1, 1)

    def outer(g, carry):
        for b in range(2):
            jj = g * 2 + b
            j = j0 + jj

            @pl.when((jj < BLK_PER_W) & (j < NBLK))
            def _():
                wait_in(b)

                @pl.when(jj >= 2)
                def _():
                    wait_out(b)

                transpose(b)
                start_in(jj + 2, b)
                pltpu.async_copy(
                    dst_v.at[b], dense_hbm.at[pl.ds(j * 64, 64)], sem_o.at[b])
        return carry

    lax.fori_loop(0, (BLK_PER_W + 1) // 2, outer, 0)

    for b in range(2):
        jj_last = BLK_PER_W - 2 + b

        @pl.when((jj_last >= 0) & (j0 + jj_last < NBLK))
        def _():
            wait_out(b)


NBUF = 2
CHUNK = 800
NCHUNK = (B // NW) // CHUNK
BPW = B // NW


@functools.partial(
    pl.kernel,
    out_type=jax.ShapeDtypeStruct((B, 128), jnp.float32),
    mesh=plsc.VectorSubcoreMesh(core_axis_name="c", subcore_axis_name="s"),
    scratch_types=[
        pltpu.VMEM((NBUF, CHUNK), jnp.int32),
        pltpu.VMEM((NBUF, CHUNK, EMBED_DIM), jnp.float32),
        pltpu.SemaphoreType.DMA((NBUF,)),
        pltpu.SemaphoreType.DMA((NBUF,)),
        pltpu.SemaphoreType.DMA((NBUF,)),
    ],
    compiler_params=pltpu.CompilerParams(use_tc_tiling_on_sc=False),
)
def _embed_gather(doc_hbm, table_hbm, out_hbm, idx_v, rows_v, sem_i, sem_g, sem_s):
    wid = lax.axis_index("s") * NC + lax.axis_index("c")
    base = wid * BPW

    def start_idx(c, b):
        pltpu.async_copy(
            doc_hbm.at[pl.ds(base + c * CHUNK, CHUNK)], idx_v.at[b], sem_i.at[b])

    def wait_idx(b):
        pltpu.make_async_copy(
            doc_hbm.at[pl.ds(0, CHUNK)], idx_v.at[b], sem_i.at[b]).wait()

    def wait_scatter(b):
        pltpu.make_async_copy(
            rows_v.at[b],
            out_hbm.at[pl.ds(0, CHUNK), pl.ds(0, EMBED_DIM)],
            sem_s.at[b]).wait()

    for b in range(NBUF):
        start_idx(b, b)

    def outer(g, carry):
        for b in range(NBUF):
            c = g * NBUF + b

            @pl.when(c >= NBUF)
            def _():
                wait_scatter(b)

            wait_idx(b)
            pltpu.async_copy(
                table_hbm.at[idx_v.at[b]], rows_v.at[b], sem_g.at[b]).wait()

            @pl.when(c + NBUF < NCHUNK)
            def _():
                start_idx(c + NBUF, b)

            pltpu.async_copy(
                rows_v.at[b],
                out_hbm.at[pl.ds(base + c * CHUNK, CHUNK), pl.ds(0, EMBED_DIM)],
                sem_s.at[b])
        return carry

    lax.fori_loop(0, NCHUNK // NBUF, outer, 0)

    for b in range(NBUF):
        wait_scatter(b)


def kernel(doc, table):
    flat = doc.reshape(B).astype(jnp.int32)
    dense = _table_transpose(table.T)
    dense_rows = dense.reshape(DENSE_ROWS * 2, EMBED_DIM)
    out = _embed_gather(flat, dense_rows)
    return out[:, :EMBED_DIM].reshape(BATCH, HIST, EMBED_DIM)
